# Initial kernel scaffold; baseline (speedup 1.0000x reference)
import jax
import jax.numpy as jnp
from jax.experimental import pallas as pl


def kernel(x, edge_index, params):
    raise NotImplementedError("write your pallas kernel here")


# speedup vs baseline: 18.6850x; 18.6850x over previous
import functools

import jax
import jax.numpy as jnp
from jax import lax
from jax.experimental import pallas as pl
from jax.experimental.pallas import tpu as pltpu
from jax.experimental.pallas import tpu_sc as plsc

N = 10000
D = 128
E_TOT = 320000 + 2 * N
NC, NS = 2, 16
NW = NC * NS
NB = 84
CHUNK = NB * 128
E_PAD = NW * CHUNK
PASSES = 4
HALF_R = 2560
ACC_R = PASSES * HALF_R
ROWS_PT = HALF_R // NS
R_BLK = 1000
N_BLK = N // R_BLK
LEAK = 0.2


def _sc_agg_body(hw_hbm, ssrc_hbm, sdst_hbm, src_hbm, dst_hbm,
                 acc_hbm, den_hbm,
                 ssrc_v, sdst_v, den_v, src_v, dst_v, ex_v, dloc_v,
                 rows_a, rows_b, mbuf, acc_sh, gsem_a, gsem_b):
    c = lax.axis_index("c")
    s = lax.axis_index("s")
    w = s * NC + c

    pltpu.sync_copy(ssrc_hbm, ssrc_v)
    pltpu.sync_copy(sdst_hbm, sdst_v)
    pltpu.sync_copy(src_hbm.at[w], src_v)
    pltpu.sync_copy(dst_hbm.at[w], dst_v)

    def mbody(i, m):
        return jnp.maximum(m, ssrc_v[pl.ds(i * 16, 16)])
    m = lax.fori_loop(0, N // 16, mbody,
                      jnp.full((16,), -3e38, jnp.float32))
    for shv in (8, 4, 2, 1):
        mbuf[pl.ds(0, 16)] = m
        mbuf[pl.ds(16, 16)] = m
        m = jnp.maximum(m, mbuf[pl.ds(shv, 16)])
    maxs = m

    def zden(i, carry):
        den_v[pl.ds(i * 16, 16)] = jnp.zeros((16,), jnp.float32)
        return carry
    lax.fori_loop(0, N // 16, zden, 0)

    base = w * CHUNK
    lane = lax.iota(jnp.int32, 16)
    def abody(b, carry):
        for u in range(8):
            sv = src_v[b, pl.ds(u * 16, 16)]
            dv = dst_v[b, pl.ds(u * 16, 16)]
            a_s = plsc.load_gather(ssrc_v, [sv])
            a_d = plsc.load_gather(sdst_v, [dv])
            ssum = a_s + a_d
            alpha = jnp.maximum(ssum, LEAK * ssum)
            sh = maxs + a_d
            shift = jnp.maximum(sh, LEAK * sh)
            ex = jnp.exp(alpha - shift)
            eid = base + b * 128 + u * 16 + lane
            ex = jnp.where(eid < E_TOT, ex, 0.0)
            ex_v[b, pl.ds(u * 16, 16)] = ex
            dl = jnp.where(dv >= 2 * HALF_R, dv - 2 * HALF_R, dv)
            dloc_v[b, pl.ds(u * 16, 16)] = jnp.where(
                dl >= HALF_R, dl - HALF_R, dl)
            plsc.addupdate_scatter(den_v, [dv], ex)
        return carry
    lax.fori_loop(0, NB, abody, 0)
    pltpu.sync_copy(den_v, den_hbm.at[w])

    def gstart(b, buf, sem):
        pltpu.async_copy(hw_hbm.at[src_v.at[b]], buf, sem)

    def gwait(b, buf, sem):
        pltpu.make_async_copy(hw_hbm.at[src_v.at[b]], buf, sem).wait()

    def zero_own_slice():
        def zrow(i, carry):
            for u in range(8):
                rows_a[i, pl.ds(u * 16, 16)] = jnp.zeros((16,), jnp.float32)
            return carry
        lax.fori_loop(0, 128, zrow, 0)
        for j in range(ROWS_PT // 80):
            pltpu.sync_copy(rows_a.at[pl.ds(0, 80)],
                            acc_sh.at[pl.ds(s * ROWS_PT + j * 80, 80)])

    for p in range(PASSES):
        lo = jnp.int32(p * HALF_R)
        hi = jnp.int32(min((p + 1) * HALF_R, N))

        def scale_scatter(b, buf, lo=lo, hi=hi):
            def tbody(t, carry):
                cfv = ex_v[b, pl.ds(t * 16, 16)]
                dvv = dst_v[b, pl.ds(t * 16, 16)]
                cfv = jnp.where((dvv >= lo) & (dvv < hi), cfv, 0.0)
                for k in range(16):
                    cfb = cfv[k]
                    r = t * 16 + k
                    for u in range(8):
                        buf[r, pl.ds(u * 16, 16)] = (
                            buf[r, pl.ds(u * 16, 16)] * cfb)
                return carry
            lax.fori_loop(0, 8, tbody, 0)
            pltpu.sync_copy(buf, acc_sh.at[dloc_v.at[b]], add=True)

        zero_own_slice()
        plsc.subcore_barrier()

        gstart(0, rows_a, gsem_a)
        def cbody(j, carry):
            b0 = 2 * j
            b1 = b0 + 1
            gstart(b1, rows_b, gsem_b)
            gwait(b0, rows_a, gsem_a)
            scale_scatter(b0, rows_a)
            @pl.when(b1 + 1 < NB)
            def _():
                gstart(b1 + 1, rows_a, gsem_a)
            gwait(b1, rows_b, gsem_b)
            scale_scatter(b1, rows_b)
            return carry
        lax.fori_loop(0, NB // 2, cbody, 0)

        plsc.subcore_barrier()
        pltpu.sync_copy(
            acc_sh.at[pl.ds(s * ROWS_PT, ROWS_PT)],
            acc_hbm.at[c, pl.ds(p * HALF_R + s * ROWS_PT, ROWS_PT)])
        plsc.subcore_barrier()


_sc_agg = functools.partial(
    pl.kernel,
    out_type=[jax.ShapeDtypeStruct((NC, ACC_R, D), jnp.float32),
              jax.ShapeDtypeStruct((NW, N), jnp.float32)],
    mesh=plsc.VectorSubcoreMesh(core_axis_name="c", subcore_axis_name="s"),
    compiler_params=pltpu.CompilerParams(needs_layout_passes=False),
    scratch_types=[
        pltpu.VMEM((N,), jnp.float32),
        pltpu.VMEM((N,), jnp.float32),
        pltpu.VMEM((N,), jnp.float32),
        pltpu.VMEM((NB, 128), jnp.int32),
        pltpu.VMEM((NB, 128), jnp.int32),
        pltpu.VMEM((NB, 128), jnp.float32),
        pltpu.VMEM((NB, 128), jnp.int32),
        pltpu.VMEM((128, D), jnp.float32),
        pltpu.VMEM((128, D), jnp.float32),
        pltpu.VMEM((128,), jnp.float32),
        pltpu.VMEM_SHARED((HALF_R, D), jnp.float32),
        pltpu.SemaphoreType.DMA,
        pltpu.SemaphoreType.DMA,
    ],
)(_sc_agg_body)


def _mm0_body(x_ref, w_ref, a_ref, hw_ref, s_ref):
    hw = jnp.dot(x_ref[...], w_ref[...], preferred_element_type=jnp.float32)
    hw_ref[...] = hw
    s_ref[...] = jnp.dot(hw, a_ref[...], preferred_element_type=jnp.float32)


def _act_body(acc_ref, den_ref, b_ref, h_ref):
    d = jnp.sum(den_ref[...], axis=1) + 1e-16
    a = acc_ref[...]
    h_ref[...] = jnp.tanh((a[0] + a[1]) / d[:, None] + b_ref[...])


def _pool_body(acc_ref, den_ref, b_ref, o_ref):
    i = pl.program_id(0)
    d = jnp.sum(den_ref[...], axis=1) + 1e-16
    a = acc_ref[...]
    part = jnp.sum((a[0] + a[1]) / d[:, None], axis=0, keepdims=True)

    @pl.when(i == 0)
    def _():
        o_ref[...] = jnp.zeros_like(o_ref)

    o_ref[...] += part

    @pl.when(i == N_BLK - 1)
    def _():
        o_ref[...] = o_ref[...] * (1.0 / N) + b_ref[...]


_mm0 = pl.pallas_call(
    _mm0_body,
    grid=(N_BLK,),
    in_specs=[pl.BlockSpec((R_BLK, D), lambda r: (r, 0)),
              pl.BlockSpec((D, D), lambda r: (0, 0)),
              pl.BlockSpec((D, D), lambda r: (0, 0))],
    out_specs=[pl.BlockSpec((R_BLK, D), lambda r: (r, 0)),
               pl.BlockSpec((R_BLK, D), lambda r: (r, 0))],
    out_shape=[jax.ShapeDtypeStruct((N, D), jnp.float32),
               jax.ShapeDtypeStruct((N, D), jnp.float32)],
)

_act = pl.pallas_call(
    _act_body,
    grid=(N_BLK,),
    in_specs=[pl.BlockSpec((NC, R_BLK, D), lambda r: (0, r, 0)),
              pl.BlockSpec((R_BLK, NW), lambda r: (r, 0)),
              pl.BlockSpec((1, D), lambda r: (0, 0))],
    out_specs=pl.BlockSpec((R_BLK, D), lambda r: (r, 0)),
    out_shape=jax.ShapeDtypeStruct((N, D), jnp.float32),
)

_pool = pl.pallas_call(
    _pool_body,
    grid=(N_BLK,),
    in_specs=[pl.BlockSpec((NC, R_BLK, D), lambda r: (0, r, 0)),
              pl.BlockSpec((R_BLK, NW), lambda r: (r, 0)),
              pl.BlockSpec((1, D), lambda r: (0, 0))],
    out_specs=pl.BlockSpec((1, D), lambda r: (0, 0)),
    out_shape=jax.ShapeDtypeStruct((1, D), jnp.float32),
)


def kernel(x, edge_index, params):
    loop = jnp.arange(N, dtype=edge_index.dtype)
    src = jnp.concatenate([edge_index[0], loop, loop])
    dst = jnp.concatenate([edge_index[1], loop, loop])
    padi = jnp.arange(E_PAD - E_TOT, dtype=jnp.int32) % N
    src3 = jnp.concatenate([src, padi]).reshape(NW, NB, 128)
    dst3 = jnp.concatenate([dst, padi]).reshape(NW, NB, 128)

    As = [jnp.zeros((D, D), jnp.float32).at[:, 0].set(a_s).at[:, 1].set(a_d)
          for (_, a_s, a_d, _b) in params]

    w_stack = jnp.stack([p[0] for p in params])
    a_stack = jnp.stack(As)
    b_stack = jnp.stack([p[3] for p in params])

    def body(carry, xs):
        h, _, _ = carry
        w_i, a_i, b_i = xs
        hw_i, sp_i = _mm0(h, w_i, a_i)
        acc_n, den_n = _sc_agg(hw_i, sp_i[:, 0], sp_i[:, 1], src3, dst3)
        den_t = den_n.T
        h_next = _act(acc_n, den_t, b_i.reshape(1, D))
        return (h_next, acc_n, den_t), None

    init = (x,
            jnp.zeros((NC, ACC_R, D), jnp.float32),
            jnp.zeros((N, NW), jnp.float32))
    (_, acc, den_t), _ = lax.scan(body, init, (w_stack, a_stack, b_stack))
    return _pool(acc, den_t, params[-1][3].reshape(1, D))


# speedup vs baseline: 18.7593x; 1.0040x over previous
import functools

import jax
import jax.numpy as jnp
from jax import lax
from jax.experimental import pallas as pl
from jax.experimental.pallas import tpu as pltpu
from jax.experimental.pallas import tpu_sc as plsc

N = 10000
D = 128
E_TOT = 320000 + 2 * N
NC, NS = 2, 16
NW = NC * NS
NB = 84
CHUNK = NB * 128
E_PAD = NW * CHUNK
PASSES = 4
HALF_R = 2560
ACC_R = PASSES * HALF_R
ROWS_PT = HALF_R // NS
R_BLK = 1000
N_BLK = N // R_BLK
LEAK = 0.2


def _sc_agg_body(hw_hbm, ssrc_hbm, sdst_hbm, src_hbm, dst_hbm,
                 acc_hbm, den_hbm,
                 ssrc_v, sdst_v, den_v, src_v, dst_v, ex_v, dloc_v,
                 rows_a, rows_b, mbuf, acc_sh, gsem_a, gsem_b):
    c = lax.axis_index("c")
    s = lax.axis_index("s")
    w = s * NC + c

    pltpu.sync_copy(ssrc_hbm, ssrc_v)
    pltpu.sync_copy(sdst_hbm, sdst_v)
    pltpu.sync_copy(src_hbm.at[w], src_v)
    pltpu.sync_copy(dst_hbm.at[w], dst_v)

    def mbody(i, m):
        return jnp.maximum(m, ssrc_v[pl.ds(i * 16, 16)])
    m = lax.fori_loop(0, N // 16, mbody,
                      jnp.full((16,), -3e38, jnp.float32))
    for shv in (8, 4, 2, 1):
        mbuf[pl.ds(0, 16)] = m
        mbuf[pl.ds(16, 16)] = m
        m = jnp.maximum(m, mbuf[pl.ds(shv, 16)])
    maxs = m

    def zden(i, carry):
        den_v[pl.ds(i * 16, 16)] = jnp.zeros((16,), jnp.float32)
        return carry
    lax.fori_loop(0, N // 16, zden, 0)

    base = w * CHUNK
    lane = lax.iota(jnp.int32, 16)
    def abody(b, carry):
        for u in range(8):
            sv = src_v[b, pl.ds(u * 16, 16)]
            dv = dst_v[b, pl.ds(u * 16, 16)]
            a_s = plsc.load_gather(ssrc_v, [sv])
            a_d = plsc.load_gather(sdst_v, [dv])
            ssum = a_s + a_d
            alpha = jnp.maximum(ssum, LEAK * ssum)
            sh = maxs + a_d
            shift = jnp.maximum(sh, LEAK * sh)
            ex = jnp.exp(alpha - shift)
            eid = base + b * 128 + u * 16 + lane
            ex = jnp.where(eid < E_TOT, ex, 0.0)
            ex_v[b, pl.ds(u * 16, 16)] = ex
            dl = jnp.where(dv >= 2 * HALF_R, dv - 2 * HALF_R, dv)
            dloc_v[b, pl.ds(u * 16, 16)] = jnp.where(
                dl >= HALF_R, dl - HALF_R, dl)
            plsc.addupdate_scatter(den_v, [dv], ex)
        return carry
    lax.fori_loop(0, NB, abody, 0)
    pltpu.sync_copy(den_v, den_hbm.at[w])

    def gstart(b, buf, sem):
        pltpu.async_copy(hw_hbm.at[src_v.at[b]], buf, sem)

    def gwait(b, buf, sem):
        pltpu.make_async_copy(hw_hbm.at[src_v.at[b]], buf, sem).wait()

    def zero_own_slice():
        def zrow(i, carry):
            for u in range(8):
                rows_a[i, pl.ds(u * 16, 16)] = jnp.zeros((16,), jnp.float32)
            return carry
        lax.fori_loop(0, 128, zrow, 0)
        for j in range(ROWS_PT // 80):
            pltpu.sync_copy(rows_a.at[pl.ds(0, 80)],
                            acc_sh.at[pl.ds(s * ROWS_PT + j * 80, 80)])

    for p in range(PASSES):
        lo = jnp.int32(p * HALF_R)
        hi = jnp.int32(min((p + 1) * HALF_R, N))

        def scale_scatter(b, buf, lo=lo, hi=hi):
            def tbody(t, carry):
                cfv = ex_v[b, pl.ds(t * 16, 16)]
                dvv = dst_v[b, pl.ds(t * 16, 16)]
                cfv = jnp.where((dvv >= lo) & (dvv < hi), cfv, 0.0)
                for k in range(16):
                    cfb = cfv[k]
                    r = t * 16 + k
                    for u in range(8):
                        buf[r, pl.ds(u * 16, 16)] = (
                            buf[r, pl.ds(u * 16, 16)] * cfb)
                return carry
            lax.fori_loop(0, 8, tbody, 0)
            pltpu.sync_copy(buf, acc_sh.at[dloc_v.at[b]], add=True)

        zero_own_slice()
        plsc.subcore_barrier()

        gstart(0, rows_a, gsem_a)
        def cbody(j, carry):
            b0 = 2 * j
            b1 = b0 + 1
            gstart(b1, rows_b, gsem_b)
            gwait(b0, rows_a, gsem_a)
            scale_scatter(b0, rows_a)
            @pl.when(b1 + 1 < NB)
            def _():
                gstart(b1 + 1, rows_a, gsem_a)
            gwait(b1, rows_b, gsem_b)
            scale_scatter(b1, rows_b)
            return carry
        lax.fori_loop(0, NB // 2, cbody, 0)

        plsc.subcore_barrier()
        pltpu.sync_copy(
            acc_sh.at[pl.ds(s * ROWS_PT, ROWS_PT)],
            acc_hbm.at[c, pl.ds(p * HALF_R + s * ROWS_PT, ROWS_PT)])
        plsc.subcore_barrier()


_sc_agg = functools.partial(
    pl.kernel,
    out_type=[jax.ShapeDtypeStruct((NC, ACC_R, D), jnp.float32),
              jax.ShapeDtypeStruct((NW, N), jnp.float32)],
    mesh=plsc.VectorSubcoreMesh(core_axis_name="c", subcore_axis_name="s"),
    compiler_params=pltpu.CompilerParams(needs_layout_passes=False),
    scratch_types=[
        pltpu.VMEM((N,), jnp.float32),
        pltpu.VMEM((N,), jnp.float32),
        pltpu.VMEM((N,), jnp.float32),
        pltpu.VMEM((NB, 128), jnp.int32),
        pltpu.VMEM((NB, 128), jnp.int32),
        pltpu.VMEM((NB, 128), jnp.float32),
        pltpu.VMEM((NB, 128), jnp.int32),
        pltpu.VMEM((128, D), jnp.float32),
        pltpu.VMEM((128, D), jnp.float32),
        pltpu.VMEM((128,), jnp.float32),
        pltpu.VMEM_SHARED((HALF_R, D), jnp.float32),
        pltpu.SemaphoreType.DMA,
        pltpu.SemaphoreType.DMA,
    ],
)(_sc_agg_body)


def _mm_body(x_ref, flag_ref, acc_ref, den_ref, b_ref, w_ref, a_ref,
             hw_ref, s_ref):
    d = jnp.sum(den_ref[...], axis=1) + 1e-16
    a = acc_ref[...]
    actv = jnp.tanh((a[0] + a[1]) / d[:, None] + b_ref[...])
    f = flag_ref[...]
    h = f * x_ref[...] + (1.0 - f) * actv
    hw = jnp.dot(h, w_ref[...], preferred_element_type=jnp.float32)
    hw_ref[...] = hw
    s_ref[...] = jnp.dot(hw, a_ref[...], preferred_element_type=jnp.float32)


def _pool_body(acc_ref, den_ref, b_ref, o_ref):
    i = pl.program_id(0)
    d = jnp.sum(den_ref[...], axis=1) + 1e-16
    a = acc_ref[...]
    part = jnp.sum((a[0] + a[1]) / d[:, None], axis=0, keepdims=True)

    @pl.when(i == 0)
    def _():
        o_ref[...] = jnp.zeros_like(o_ref)

    o_ref[...] += part

    @pl.when(i == N_BLK - 1)
    def _():
        o_ref[...] = o_ref[...] * (1.0 / N) + b_ref[...]


_mm = pl.pallas_call(
    _mm_body,
    grid=(N_BLK,),
    in_specs=[pl.BlockSpec((R_BLK, D), lambda r: (r, 0)),
              pl.BlockSpec((1, D), lambda r: (0, 0)),
              pl.BlockSpec((NC, R_BLK, D), lambda r: (0, r, 0)),
              pl.BlockSpec((R_BLK, NW), lambda r: (r, 0)),
              pl.BlockSpec((1, D), lambda r: (0, 0)),
              pl.BlockSpec((D, D), lambda r: (0, 0)),
              pl.BlockSpec((D, D), lambda r: (0, 0))],
    out_specs=[pl.BlockSpec((R_BLK, D), lambda r: (r, 0)),
               pl.BlockSpec((R_BLK, D), lambda r: (r, 0))],
    out_shape=[jax.ShapeDtypeStruct((N, D), jnp.float32),
               jax.ShapeDtypeStruct((N, D), jnp.float32)],
)

_pool = pl.pallas_call(
    _pool_body,
    grid=(N_BLK,),
    in_specs=[pl.BlockSpec((NC, R_BLK, D), lambda r: (0, r, 0)),
              pl.BlockSpec((R_BLK, NW), lambda r: (r, 0)),
              pl.BlockSpec((1, D), lambda r: (0, 0))],
    out_specs=pl.BlockSpec((1, D), lambda r: (0, 0)),
    out_shape=jax.ShapeDtypeStruct((1, D), jnp.float32),
)


def kernel(x, edge_index, params):
    loop = jnp.arange(N, dtype=edge_index.dtype)
    src = jnp.concatenate([edge_index[0], loop, loop])
    dst = jnp.concatenate([edge_index[1], loop, loop])
    padi = jnp.arange(E_PAD - E_TOT, dtype=jnp.int32) % N
    src3 = jnp.concatenate([src, padi]).reshape(NW, NB, 128)
    dst3 = jnp.concatenate([dst, padi]).reshape(NW, NB, 128)

    As = [jnp.zeros((D, D), jnp.float32).at[:, 0].set(a_s).at[:, 1].set(a_d)
          for (_, a_s, a_d, _b) in params]

    nl = len(params)
    w_stack = jnp.stack([p[0] for p in params])
    a_stack = jnp.stack(As)
    bprev_stack = jnp.stack(
        [jnp.zeros((1, D), jnp.float32)]
        + [params[i][3].reshape(1, D) for i in range(nl - 1)])
    flag_stack = jnp.concatenate(
        [jnp.ones((1, 1, D), jnp.float32),
         jnp.zeros((nl - 1, 1, D), jnp.float32)])

    def body(carry, xs):
        acc_c, den_c = carry
        w_i, a_i, bp_i, f_i = xs
        hw_i, sp_i = _mm(x, f_i, acc_c, den_c, bp_i, w_i, a_i)
        acc_n, den_n = _sc_agg(hw_i, sp_i[:, 0], sp_i[:, 1], src3, dst3)
        return (acc_n, den_n.T), None

    init = (jnp.zeros((NC, ACC_R, D), jnp.float32),
            jnp.zeros((N, NW), jnp.float32))
    (acc, den_t), _ = lax.scan(
        body, init, (w_stack, a_stack, bprev_stack, flag_stack))
    return _pool(acc, den_t, params[-1][3].reshape(1, D))
